# baseline (speedup 1.0000x reference)
import functools

import jax
import jax.numpy as jnp
from jax import lax
from jax.experimental import pallas as pl
from jax.experimental.pallas import tpu as pltpu
from jax.experimental.pallas import tpu_sc as plsc

B = 4096
F = 26
D = 64
LANES = 16
FPAD = 32
NUM_WORKERS = 32
DIMS_PER_W = D // NUM_WORKERS
NSUBC = 16
V = 100000
BC = 256
NCH = B // BC
NIB = 2


def _emb_body(idx_hbm, tab_hbm, out_hbm, row_v, idx_v, outc_v,
              sh_idx, sem0, sem1, semr, semp):
    sems = (sem0, sem1)
    cid = lax.axis_index("c")
    sid = lax.axis_index("s")
    wid = sid * 2 + cid
    d0 = wid * DIMS_PER_W

    pltpu.make_async_copy(tab_hbm.at[d0], row_v, semr).start()
    stripe = pl.ds(sid * (B // NSUBC), B // NSUBC)
    pltpu.sync_copy(idx_hbm.at[:, stripe], sh_idx.at[:, stripe])
    plsc.subcore_barrier()

    def load_idx(c, buf):
        pltpu.make_async_copy(
            sh_idx.at[:, pl.ds(c * BC, BC)], idx_v.at[buf], sems[buf]
        ).start()

    def wait_idx(buf):
        pltpu.make_async_copy(
            sh_idx.at[:, pl.ds(0, BC)], idx_v.at[buf], sems[buf]
        ).wait()

    def run_dim(d):
        def compute_chunk(c, buf):
            @plsc.parallel_loop(0, BC // LANES, unroll=2)
            def jbody(j):
                sl = pl.ds(j * LANES, LANES)
                vals = [
                    plsc.load_gather(row_v, [idx_v[buf, f, sl]])
                    for f in range(F)
                ]
                while len(vals) > 1:
                    nxt = [a + b for a, b in zip(vals[0::2], vals[1::2])]
                    if len(vals) % 2:
                        nxt.append(vals[-1])
                    vals = nxt
                outc_v[sl] = vals[0]
            pltpu.sync_copy(outc_v, out_hbm.at[d, pl.ds(c * BC, BC)])

        load_idx(0, 0)

        def outer(it, carry):
            c2 = it * NIB
            for b in range(NIB):
                c = c2 + b

                @pl.when(c + 1 < NCH)
                def _():
                    load_idx(c + 1, 1 - b)

                wait_idx(b)
                compute_chunk(c, b)
            return carry

        lax.fori_loop(0, NCH // NIB, outer, 0)

    pltpu.make_async_copy(tab_hbm.at[d0], row_v, semr).wait()
    run_dim(d0)
    pltpu.make_async_copy(tab_hbm.at[d0 + 1], row_v, semp).start()
    pltpu.make_async_copy(tab_hbm.at[d0 + 1], row_v, semp).wait()
    run_dim(d0 + 1)


def kernel(x, table):
    xt = jnp.pad(x.T, ((0, FPAD - F), (0, 0)))
    tt = table.T
    mesh = plsc.VectorSubcoreMesh(core_axis_name="c", subcore_axis_name="s")
    k = functools.partial(
        pl.kernel,
        mesh=mesh,
        out_type=jax.ShapeDtypeStruct((D, B), jnp.float32),
        scratch_types=[
            pltpu.VMEM((V,), jnp.float32),
            pltpu.VMEM((NIB, FPAD, BC), jnp.int32),
            pltpu.VMEM((BC,), jnp.float32),
            pltpu.VMEM_SHARED((FPAD, B), jnp.int32),
            pltpu.SemaphoreType.DMA,
            pltpu.SemaphoreType.DMA,
            pltpu.SemaphoreType.DMA,
            pltpu.SemaphoreType.DMA,
        ],
        compiler_params=pltpu.CompilerParams(
            use_tc_tiling_on_sc=True, needs_layout_passes=False
        ),
    )(_emb_body)
    out_t = k(xt, tt)
    return out_t.T


# speedup vs baseline: 1.0448x; 1.0448x over previous
import functools

import jax
import jax.numpy as jnp
from jax import lax
from jax.experimental import pallas as pl
from jax.experimental.pallas import tpu as pltpu
from jax.experimental.pallas import tpu_sc as plsc

B = 4096
F = 26
D = 64
LANES = 16
FPAD = 32
NUM_WORKERS = 32
DIMS_PER_W = D // NUM_WORKERS
NSUBC = 16
V = 100000
BC = 256
NCH = B // BC
NIB = 2


def _emb_body(idx_hbm, tab_hbm, out_hbm, row_v, idx_v, outc_v,
              sh_idx, sem0, sem1, semr, semp):
    sems = (sem0, sem1)
    cid = lax.axis_index("c")
    sid = lax.axis_index("s")
    wid = sid * 2 + cid
    d0 = wid * DIMS_PER_W

    pltpu.make_async_copy(tab_hbm.at[d0], row_v, semr).start()
    stripe = pl.ds(sid * (B // NSUBC), B // NSUBC)
    pltpu.sync_copy(idx_hbm.at[:, stripe], sh_idx.at[:, stripe])
    plsc.subcore_barrier()

    def load_idx(c, buf):
        pltpu.make_async_copy(
            sh_idx.at[:, pl.ds(c * BC, BC)], idx_v.at[buf], sems[buf]
        ).start()

    def wait_idx(buf):
        pltpu.make_async_copy(
            sh_idx.at[:, pl.ds(0, BC)], idx_v.at[buf], sems[buf]
        ).wait()

    def run_dim(d):
        def compute_chunk(c, buf):
            def jbody(j, carry):
                sl = pl.ds(j * LANES, LANES)
                vals = [
                    plsc.load_gather(row_v, [idx_v[buf, f, sl]])
                    for f in range(F)
                ]
                while len(vals) > 1:
                    nxt = [a + b for a, b in zip(vals[0::2], vals[1::2])]
                    if len(vals) % 2:
                        nxt.append(vals[-1])
                    vals = nxt
                outc_v[sl] = vals[0]
                return carry

            lax.fori_loop(0, BC // LANES, jbody, 0)
            pltpu.sync_copy(outc_v, out_hbm.at[d, pl.ds(c * BC, BC)])

        load_idx(0, 0)

        def outer(it, carry):
            c2 = it * NIB
            for b in range(NIB):
                c = c2 + b

                @pl.when(c + 1 < NCH)
                def _():
                    load_idx(c + 1, 1 - b)

                wait_idx(b)
                compute_chunk(c, b)
            return carry

        lax.fori_loop(0, NCH // NIB, outer, 0)

    pltpu.make_async_copy(tab_hbm.at[d0], row_v, semr).wait()
    run_dim(d0)
    pltpu.make_async_copy(tab_hbm.at[d0 + 1], row_v, semp).start()
    pltpu.make_async_copy(tab_hbm.at[d0 + 1], row_v, semp).wait()
    run_dim(d0 + 1)


def kernel(x, table):
    xt = jnp.pad(x.T, ((0, FPAD - F), (0, 0)))
    tt = table.T
    mesh = plsc.VectorSubcoreMesh(core_axis_name="c", subcore_axis_name="s")
    k = functools.partial(
        pl.kernel,
        mesh=mesh,
        out_type=jax.ShapeDtypeStruct((D, B), jnp.float32),
        scratch_types=[
            pltpu.VMEM((V,), jnp.float32),
            pltpu.VMEM((NIB, FPAD, BC), jnp.int32),
            pltpu.VMEM((BC,), jnp.float32),
            pltpu.VMEM_SHARED((FPAD, B), jnp.int32),
            pltpu.SemaphoreType.DMA,
            pltpu.SemaphoreType.DMA,
            pltpu.SemaphoreType.DMA,
            pltpu.SemaphoreType.DMA,
        ],
        compiler_params=pltpu.CompilerParams(
            use_tc_tiling_on_sc=True, needs_layout_passes=False
        ),
    )(_emb_body)
    out_t = k(xt, tt)
    return out_t.T


# speedup vs baseline: 1.0891x; 1.0425x over previous
import functools

import jax
import jax.numpy as jnp
from jax import lax
from jax.experimental import pallas as pl
from jax.experimental.pallas import tpu as pltpu
from jax.experimental.pallas import tpu_sc as plsc

B = 4096
F = 26
D = 64
LANES = 16
FPAD = 32
NUM_WORKERS = 32
DIMS_PER_W = D // NUM_WORKERS
NSUBC = 16
V = 100000
BC = 256
NCH = B // BC
NIB = 2


def _emb_body(idx_hbm, tab_hbm, out_hbm, row_v, idx_v, outc_v,
              sh_idx, sem0, sem1, semr, semp, semo):
    sems = (sem0, sem1)
    cid = lax.axis_index("c")
    sid = lax.axis_index("s")
    wid = sid * 2 + cid
    d0 = wid * DIMS_PER_W

    pltpu.make_async_copy(tab_hbm.at[d0], row_v, semr).start()
    stripe = pl.ds(sid * (B // NSUBC), B // NSUBC)
    pltpu.sync_copy(idx_hbm.at[:, stripe], sh_idx.at[:, stripe])
    plsc.subcore_barrier()

    def load_idx(c, buf):
        pltpu.make_async_copy(
            sh_idx.at[:, pl.ds(c * BC, BC)], idx_v.at[buf], sems[buf]
        ).start()

    def wait_idx(buf):
        pltpu.make_async_copy(
            sh_idx.at[:, pl.ds(0, BC)], idx_v.at[buf], sems[buf]
        ).wait()

    def run_dim(d):
        def compute_chunk(c, buf):
            def jbody(j, carry):
                sl = pl.ds(j * LANES, LANES)
                vals = [
                    plsc.load_gather(row_v, [idx_v[buf, f, sl]])
                    for f in range(F)
                ]
                while len(vals) > 1:
                    nxt = [a + b for a, b in zip(vals[0::2], vals[1::2])]
                    if len(vals) % 2:
                        nxt.append(vals[-1])
                    vals = nxt
                outc_v[buf, sl] = vals[0]
                return carry

            lax.fori_loop(0, BC // LANES, jbody, 0)
            pltpu.make_async_copy(
                outc_v.at[buf], out_hbm.at[d, pl.ds(c * BC, BC)], semo
            ).start()

        load_idx(0, 0)

        def outer(it, carry):
            c2 = it * NIB
            for b in range(NIB):
                c = c2 + b

                @pl.when(c + 1 < NCH)
                def _():
                    load_idx(c + 1, 1 - b)

                wait_idx(b)

                @pl.when(c >= NIB)
                def _():
                    pltpu.make_async_copy(
                        outc_v.at[b], out_hbm.at[d, pl.ds(0, BC)], semo
                    ).wait()

                compute_chunk(c, b)
            return carry

        lax.fori_loop(0, NCH // NIB, outer, 0)
        for b in range(NIB):
            pltpu.make_async_copy(
                outc_v.at[b], out_hbm.at[d, pl.ds(0, BC)], semo
            ).wait()

    pltpu.make_async_copy(tab_hbm.at[d0], row_v, semr).wait()
    run_dim(d0)
    pltpu.make_async_copy(tab_hbm.at[d0 + 1], row_v, semp).start()
    pltpu.make_async_copy(tab_hbm.at[d0 + 1], row_v, semp).wait()
    run_dim(d0 + 1)


def kernel(x, table):
    xt = jnp.pad(x.T, ((0, FPAD - F), (0, 0)))
    tt = table.T
    mesh = plsc.VectorSubcoreMesh(core_axis_name="c", subcore_axis_name="s")
    k = functools.partial(
        pl.kernel,
        mesh=mesh,
        out_type=jax.ShapeDtypeStruct((D, B), jnp.float32),
        scratch_types=[
            pltpu.VMEM((V,), jnp.float32),
            pltpu.VMEM((NIB, FPAD, BC), jnp.int32),
            pltpu.VMEM((NIB, BC), jnp.float32),
            pltpu.VMEM_SHARED((FPAD, B), jnp.int32),
            pltpu.SemaphoreType.DMA,
            pltpu.SemaphoreType.DMA,
            pltpu.SemaphoreType.DMA,
            pltpu.SemaphoreType.DMA,
            pltpu.SemaphoreType.DMA,
        ],
        compiler_params=pltpu.CompilerParams(
            use_tc_tiling_on_sc=True, needs_layout_passes=False
        ),
    )(_emb_body)
    out_t = k(xt, tt)
    return out_t.T
